# baseline (speedup 1.0000x reference)
import jax
import jax.numpy as jnp
from jax.experimental import pallas as pl
from jax.experimental.pallas import tpu as pltpu

_N = 10000
_NIN = 128
_NOUT = 127
_BM = 200


def _gcn_kernel(adj_ref, xb_ref, xr_ref, w_ref, b_ref, o_ref):
    a = adj_ref[...].astype(jnp.bfloat16)
    acc = jnp.dot(a, xb_ref[...], preferred_element_type=jnp.float32)
    h = jnp.dot(
        acc.astype(jnp.bfloat16),
        w_ref[...].astype(jnp.bfloat16),
        preferred_element_type=jnp.float32,
    )
    h = jnp.maximum(h + b_ref[...], 0.0)
    o_ref[...] = (h + xr_ref[...])[:, :_NOUT]


def kernel(x, adj, W, b):
    w_p = jnp.pad(W, ((0, 0), (0, _NIN - _NOUT)))
    b_p = jnp.pad(b, (0, _NIN - _NOUT)).reshape(1, _NIN)
    x_b = x.astype(jnp.bfloat16)
    return pl.pallas_call(
        _gcn_kernel,
        grid=(_N // _BM,),
        in_specs=[
            pl.BlockSpec((_BM, _N), lambda i: (i, 0)),
            pl.BlockSpec((_N, _NIN), lambda i: (0, 0)),
            pl.BlockSpec((_BM, _NIN), lambda i: (i, 0)),
            pl.BlockSpec((_NIN, _NIN), lambda i: (0, 0)),
            pl.BlockSpec((1, _NIN), lambda i: (0, 0)),
        ],
        out_specs=pl.BlockSpec((_BM, _NOUT), lambda i: (i, 0)),
        out_shape=jax.ShapeDtypeStruct((_N, _NOUT), jnp.float32),
        compiler_params=pltpu.CompilerParams(
            dimension_semantics=("arbitrary",),
        ),
    )(adj, x_b, x, w_p, b_p)


# speedup vs baseline: 1.0379x; 1.0379x over previous
import jax
import jax.numpy as jnp
from jax.experimental import pallas as pl
from jax.experimental.pallas import tpu as pltpu

_N = 10000
_NIN = 128
_NOUT = 127
_BM = 200


def _gcn_kernel(adj_ref, x_ref, w_ref, b_ref, o_ref, xb_ref):
    i = pl.program_id(0)

    @pl.when(i == 0)
    def _cast_x_once():
        xb_ref[...] = x_ref[...].astype(jnp.bfloat16)

    a = adj_ref[...].astype(jnp.bfloat16)
    acc = jnp.dot(a, xb_ref[...], preferred_element_type=jnp.float32)
    h = jnp.dot(
        acc.astype(jnp.bfloat16),
        w_ref[...],
        preferred_element_type=jnp.float32,
    )
    h = jnp.maximum(h + b_ref[...], 0.0)
    res = x_ref[pl.ds(i * _BM, _BM), :]
    o_ref[...] = (h + res)[:, :_NOUT]


def kernel(x, adj, W, b):
    w_p = jnp.pad(W, ((0, 0), (0, _NIN - _NOUT))).astype(jnp.bfloat16)
    b_p = jnp.pad(b, (0, _NIN - _NOUT)).reshape(1, _NIN)
    return pl.pallas_call(
        _gcn_kernel,
        grid=(_N // _BM,),
        in_specs=[
            pl.BlockSpec((_BM, _N), lambda i: (i, 0)),
            pl.BlockSpec((_N, _NIN), lambda i: (0, 0)),
            pl.BlockSpec((_NIN, _NIN), lambda i: (0, 0)),
            pl.BlockSpec((1, _NIN), lambda i: (0, 0)),
        ],
        out_specs=pl.BlockSpec((_BM, _NOUT), lambda i: (i, 0)),
        out_shape=jax.ShapeDtypeStruct((_N, _NOUT), jnp.float32),
        scratch_shapes=[pltpu.VMEM((_N, _NIN), jnp.bfloat16)],
        compiler_params=pltpu.CompilerParams(
            dimension_semantics=("arbitrary",),
        ),
    )(adj, x, w_p, b_p)


# speedup vs baseline: 1.0721x; 1.0329x over previous
import jax
import jax.numpy as jnp
from jax.experimental import pallas as pl
from jax.experimental.pallas import tpu as pltpu

_N = 10000
_NIN = 128
_NOUT = 127
_BM = 400


def _gcn_kernel(adj_ref, x_ref, w_ref, b_ref, o_ref, xb_ref):
    i = pl.program_id(0)

    @pl.when(i == 0)
    def _cast_x_once():
        xb_ref[...] = x_ref[...].astype(jnp.bfloat16)

    a = adj_ref[...].astype(jnp.bfloat16)
    acc = jnp.dot(a, xb_ref[...], preferred_element_type=jnp.float32)
    h = jnp.dot(
        acc.astype(jnp.bfloat16),
        w_ref[...],
        preferred_element_type=jnp.float32,
    )
    h = jnp.maximum(h + b_ref[...], 0.0)
    res = x_ref[pl.ds(i * _BM, _BM), :]
    o_ref[...] = (h + res)[:, :_NOUT]


def kernel(x, adj, W, b):
    w_p = jnp.pad(W, ((0, 0), (0, _NIN - _NOUT))).astype(jnp.bfloat16)
    b_p = jnp.pad(b, (0, _NIN - _NOUT)).reshape(1, _NIN)
    return pl.pallas_call(
        _gcn_kernel,
        grid=(_N // _BM,),
        in_specs=[
            pl.BlockSpec((_BM, _N), lambda i: (i, 0)),
            pl.BlockSpec((_N, _NIN), lambda i: (0, 0)),
            pl.BlockSpec((_NIN, _NIN), lambda i: (0, 0)),
            pl.BlockSpec((1, _NIN), lambda i: (0, 0)),
        ],
        out_specs=pl.BlockSpec((_BM, _NOUT), lambda i: (i, 0)),
        out_shape=jax.ShapeDtypeStruct((_N, _NOUT), jnp.float32),
        scratch_shapes=[pltpu.VMEM((_N, _NIN), jnp.bfloat16)],
        compiler_params=pltpu.CompilerParams(
            dimension_semantics=("arbitrary",),
        ),
    )(adj, x, w_p, b_p)
